# baseline (speedup 1.0000x reference)
import functools

import jax
import jax.numpy as jnp
from jax.experimental import pallas as pl

_K = 32


def _edge_conv(x, src, dst, W, b, n):
    h = jnp.concatenate([x[dst], x[src] - x[dst]], axis=1) @ W + b
    out = jax.ops.segment_max(h, dst, num_segments=n)
    return jnp.where(jnp.isneginf(out), 0.0, out)


def _gcn_conv(x, src, dst, W, b, n):
    loop = jnp.arange(n, dtype=src.dtype)
    s = jnp.concatenate([src, loop])
    d = jnp.concatenate([dst, loop])
    deg = jnp.zeros((n,), x.dtype).at[d].add(1.0)
    dinv = jnp.where(deg > 0, 1.0 / jnp.sqrt(deg), 0.0)
    norm = dinv[s] * dinv[d]
    h = x @ W
    out = jnp.zeros((n, W.shape[1]), x.dtype).at[d].add(norm[:, None] * h[s])
    return out + b


def _bn_relu_add_block(h_ref, stats_ref, idn_ref, out_ref):
    scale = stats_ref[0, :]
    shift = stats_ref[1, :]
    h = h_ref[...] * scale[None, :] + shift[None, :]
    out_ref[...] = jnp.maximum(h, 0.0) + idn_ref[...]


def _bn_relu_add(h, identity, gamma, beta, eps=1e-5):
    n, c = h.shape
    mu = jnp.mean(h, axis=0)
    var = jnp.mean((h - mu) ** 2, axis=0)
    rstd = 1.0 / jnp.sqrt(var + eps)
    stats = jnp.stack([gamma * rstd, beta - gamma * mu * rstd])
    blk = 1000
    grid = (n + blk - 1) // blk
    return pl.pallas_call(
        _bn_relu_add_block,
        out_shape=jax.ShapeDtypeStruct((n, c), h.dtype),
        grid=(grid,),
        in_specs=[
            pl.BlockSpec((blk, c), lambda i: (i, 0)),
            pl.BlockSpec((2, c), lambda i: (0, 0)),
            pl.BlockSpec((blk, c), lambda i: (i, 0)),
        ],
        out_specs=pl.BlockSpec((blk, c), lambda i: (i, 0)),
    )(h, stats, identity)


def _conv_res_block(x, src, dst, p, n):
    identity = x @ p["Wr"] + p["br"] if "Wr" in p else x
    h = _gcn_conv(x, src, dst, p["Wg"], p["bg"], n)
    return _bn_relu_add(h, identity, p["gamma"], p["beta"])


def _knn_indices(pos_h, pos_l, k, chunk=2500):
    idxs = []
    for i in range(0, pos_h.shape[0], chunk):
        q = pos_h[i:i + chunk]
        dist = jnp.sum((q[:, None, :] - pos_l[None, :, :]) ** 2, axis=-1)
        _, ind = jax.lax.top_k(-dist, k)
        idxs.append(ind)
    return jnp.concatenate(idxs, axis=0)


def _knn_interpolate(x, pos_l, pos_h, k=_K):
    idx = _knn_indices(pos_h, pos_l, k)
    diff = pos_h[:, None, :] - pos_l[idx]
    sq = jnp.sum(diff * diff, axis=-1)
    w = 1.0 / jnp.maximum(sq, 1e-16)
    num = jnp.sum(w[:, :, None] * x[idx], axis=1)
    den = jnp.sum(w, axis=1, keepdims=True)
    return num / den


def _cfd_error(pos_l, src, dst, p, n):
    x = _edge_conv(pos_l, src, dst, p["W0"], p["b0"], n)
    x1 = _edge_conv(x, src, dst, p["W1"], p["b1"], n)
    x2 = _edge_conv(x, src, dst, p["W2"], p["b2"], n)
    x2 = x2 * x2
    t1 = _edge_conv(x, src, dst, p["W3a"], p["b3a"], n)
    t2 = _edge_conv(x, src, dst, p["W3b"], p["b3b"], n)
    x4 = t1 * t1 + t2 * t2
    xc = jnp.concatenate([x1, x2, x4], axis=1)
    return _edge_conv(xc, src, dst, p["W4"], p["b4"], n)


def kernel(x_l, pos_l, pos_h, params, edge_index_l, edge_index_h):
    n_l = x_l.shape[0]
    n_h = pos_h.shape[0]
    sl, dl = edge_index_l[0], edge_index_l[1]
    sh, dh = edge_index_h[0], edge_index_h[1]
    e = _cfd_error(pos_l, sl, dl, params["err"], n_l)
    x = jnp.concatenate([x_l, e], axis=1)
    x = _knn_interpolate(x, pos_l, pos_h, _K)
    x1 = _conv_res_block(x, sh, dh, params["enc1"], n_h)
    x2 = _conv_res_block(x1, sh, dh, params["enc2"], n_h)
    m = _gcn_conv(x2, sh, dh, params["mp"]["Wg"], params["mp"]["bg"], n_h)
    dec_in = x1 + m
    d1 = _conv_res_block(dec_in, sh, dh, params["dec1"], n_h)
    d2 = _conv_res_block(d1, sh, dh, params["dec2"], n_h)
    return d2


# speedup vs baseline: 2.7370x; 2.7370x over previous
import functools

import jax
import jax.numpy as jnp
from jax.experimental import pallas as pl
from jax.experimental.pallas import tpu as pltpu

_K = 32


def _edge_conv(x, src, dst, W, b, n):
    h = jnp.concatenate([x[dst], x[src] - x[dst]], axis=1) @ W + b
    out = jax.ops.segment_max(h, dst, num_segments=n)
    return jnp.where(jnp.isneginf(out), 0.0, out)


def _gcn_conv(x, src, dst, W, b, n):
    loop = jnp.arange(n, dtype=src.dtype)
    s = jnp.concatenate([src, loop])
    d = jnp.concatenate([dst, loop])
    deg = jnp.zeros((n,), x.dtype).at[d].add(1.0)
    dinv = jnp.where(deg > 0, 1.0 / jnp.sqrt(deg), 0.0)
    norm = dinv[s] * dinv[d]
    h = x @ W
    out = jnp.zeros((n, W.shape[1]), x.dtype).at[d].add(norm[:, None] * h[s])
    return out + b


def _bn_relu_add_block(h_ref, stats_ref, idn_ref, out_ref):
    scale = stats_ref[0, :]
    shift = stats_ref[1, :]
    h = h_ref[...] * scale[None, :] + shift[None, :]
    out_ref[...] = jnp.maximum(h, 0.0) + idn_ref[...]


def _bn_relu_add(h, identity, gamma, beta, eps=1e-5):
    n, c = h.shape
    mu = jnp.mean(h, axis=0)
    var = jnp.mean((h - mu) ** 2, axis=0)
    rstd = 1.0 / jnp.sqrt(var + eps)
    stats = jnp.stack([gamma * rstd, beta - gamma * mu * rstd])
    blk = 1000
    grid = (n + blk - 1) // blk
    return pl.pallas_call(
        _bn_relu_add_block,
        out_shape=jax.ShapeDtypeStruct((n, c), h.dtype),
        grid=(grid,),
        in_specs=[
            pl.BlockSpec((blk, c), lambda i: (i, 0)),
            pl.BlockSpec((2, c), lambda i: (0, 0)),
            pl.BlockSpec((blk, c), lambda i: (i, 0)),
        ],
        out_specs=pl.BlockSpec((blk, c), lambda i: (i, 0)),
    )(h, stats, identity)


def _conv_res_block(x, src, dst, p, n):
    identity = x @ p["Wr"] + p["br"] if "Wr" in p else x
    h = _gcn_conv(x, src, dst, p["Wg"], p["bg"], n)
    return _bn_relu_add(h, identity, p["gamma"], p["beta"])


def _knn_block(posh_ref, plT_ref, feat_ref, out_ref, orig_ref, work_ref):
    qx = posh_ref[:, 0:1]
    qy = posh_ref[:, 1:2]
    px = plT_ref[0:1, :]
    py = plT_ref[1:2, :]
    dx = qx - px
    dy = qy - py
    d = dx * dx + dy * dy
    orig_ref[...] = d
    work_ref[...] = d

    def body(i, _):
        dw = work_ref[...]
        m = jnp.min(dw, axis=1, keepdims=True)
        work_ref[...] = jnp.where(dw <= m, jnp.float32(jnp.inf), dw)
        return m

    t = jax.lax.fori_loop(0, _K, body, jnp.zeros((posh_ref.shape[0], 1), jnp.float32))
    dorig = orig_ref[...]
    w = jnp.where(dorig <= t, 1.0 / jnp.maximum(dorig, 1e-16), 0.0)
    r = jnp.dot(w, feat_ref[...], preferred_element_type=jnp.float32,
                precision=jax.lax.Precision.HIGHEST)
    out_ref[...] = r / r[:, 6:7]


def _knn_interpolate(x, pos_l, pos_h, k=_K):
    n_h = pos_h.shape[0]
    n_l = pos_l.shape[0]
    npad = ((n_l + 127) // 128) * 128 + 128
    plT = jnp.full((8, npad), 1e9, dtype=jnp.float32)
    plT = plT.at[0, :n_l].set(pos_l[:, 0]).at[1, :n_l].set(pos_l[:, 1])
    feat = jnp.zeros((npad, 8), dtype=jnp.float32)
    feat = feat.at[:n_l, :x.shape[1]].set(x).at[:n_l, 6].set(1.0)
    qblk = 200
    grid = n_h // qblk
    out = pl.pallas_call(
        _knn_block,
        out_shape=jax.ShapeDtypeStruct((n_h, 8), jnp.float32),
        grid=(grid,),
        in_specs=[
            pl.BlockSpec((qblk, 2), lambda i: (i, 0)),
            pl.BlockSpec((8, npad), lambda i: (0, 0)),
            pl.BlockSpec((npad, 8), lambda i: (0, 0)),
        ],
        out_specs=pl.BlockSpec((qblk, 8), lambda i: (i, 0)),
        scratch_shapes=[
            pltpu.VMEM((qblk, npad), jnp.float32),
            pltpu.VMEM((qblk, npad), jnp.float32),
        ],
    )(pos_h, plT, feat)
    return out[:, :6]


def _cfd_error(pos_l, src, dst, p, n):
    x = _edge_conv(pos_l, src, dst, p["W0"], p["b0"], n)
    x1 = _edge_conv(x, src, dst, p["W1"], p["b1"], n)
    x2 = _edge_conv(x, src, dst, p["W2"], p["b2"], n)
    x2 = x2 * x2
    t1 = _edge_conv(x, src, dst, p["W3a"], p["b3a"], n)
    t2 = _edge_conv(x, src, dst, p["W3b"], p["b3b"], n)
    x4 = t1 * t1 + t2 * t2
    xc = jnp.concatenate([x1, x2, x4], axis=1)
    return _edge_conv(xc, src, dst, p["W4"], p["b4"], n)


def kernel(x_l, pos_l, pos_h, params, edge_index_l, edge_index_h):
    n_l = x_l.shape[0]
    n_h = pos_h.shape[0]
    sl, dl = edge_index_l[0], edge_index_l[1]
    sh, dh = edge_index_h[0], edge_index_h[1]
    e = _cfd_error(pos_l, sl, dl, params["err"], n_l)
    x = jnp.concatenate([x_l, e], axis=1)
    x = _knn_interpolate(x, pos_l, pos_h, _K)
    x1 = _conv_res_block(x, sh, dh, params["enc1"], n_h)
    x2 = _conv_res_block(x1, sh, dh, params["enc2"], n_h)
    m = _gcn_conv(x2, sh, dh, params["mp"]["Wg"], params["mp"]["bg"], n_h)
    dec_in = x1 + m
    d1 = _conv_res_block(dec_in, sh, dh, params["dec1"], n_h)
    d2 = _conv_res_block(d1, sh, dh, params["dec2"], n_h)
    return d2


# speedup vs baseline: 8.0922x; 2.9566x over previous
import functools

import jax
import jax.numpy as jnp
from jax import lax
from jax.experimental import pallas as pl
from jax.experimental.pallas import tpu as pltpu
from jax.experimental.pallas import tpu_sc as plsc

_K = 32


def _sc_seg_sum(table, srcs, dsts, zeros_half):
    w = table.shape[1]
    ec = srcs.shape[0] // 2
    ept = ec // 16
    ch = ept // 25
    n_pad = zeros_half.shape[0]
    stripe = n_pad // 16
    mesh = plsc.VectorSubcoreMesh(core_axis_name="c", subcore_axis_name="s")

    @functools.partial(
        pl.kernel, mesh=mesh,
        out_type=jax.ShapeDtypeStruct((2 * n_pad, w), jnp.float32),
        compiler_params=pltpu.CompilerParams(use_tc_tiling_on_sc=False),
        scratch_types=[
            pltpu.VMEM((ch,), jnp.int32),
            pltpu.VMEM((ch,), jnp.int32),
            pltpu.VMEM((ch, w), jnp.float32),
            pltpu.VMEM_SHARED((n_pad, w), jnp.float32),
            pltpu.SemaphoreType.DMA,
        ],
    )
    def body(t_hbm, s_hbm, d_hbm, z_hbm, out_hbm, sidx, didx, rows, acc, sem):
        cid = lax.axis_index("c")
        tid = lax.axis_index("s")
        pltpu.sync_copy(z_hbm.at[pl.ds(tid * stripe, stripe)],
                        acc.at[pl.ds(tid * stripe, stripe)])
        plsc.subcore_barrier()

        def step(i, carry):
            off = cid * ec + tid * ept + i * ch
            pltpu.sync_copy(s_hbm.at[pl.ds(off, ch)], sidx)
            pltpu.sync_copy(d_hbm.at[pl.ds(off, ch)], didx)
            pltpu.async_copy(t_hbm.at[sidx], rows, sem).wait()
            pltpu.sync_copy(rows, acc.at[didx], add=True)
            return carry

        lax.fori_loop(0, ept // ch, step, 0)
        plsc.subcore_barrier()
        pltpu.sync_copy(acc.at[pl.ds(tid * stripe, stripe)],
                        out_hbm.at[pl.ds(cid * n_pad + tid * stripe, stripe)])

    return body(table, srcs, dsts, zeros_half)


def _edge_conv(x, src, dst, W, b, n):
    h = jnp.concatenate([x[dst], x[src] - x[dst]], axis=1) @ W + b
    out = jax.ops.segment_max(h, dst, num_segments=n)
    return jnp.where(jnp.isneginf(out), 0.0, out)


def _gcn_aux(src, dst, n):
    deg = jnp.zeros((n,), jnp.float32).at[dst].add(1.0) + 1.0
    dinv = 1.0 / jnp.sqrt(deg)
    n_pad = ((n + 127) // 128) * 128
    srcs_col = jnp.concatenate([src, src + n])
    dsts_col = jnp.concatenate([dst, dst])
    z16 = jnp.zeros((n_pad, 16), jnp.float32)
    return dict(dinv=dinv, n_pad=n_pad, srcs_col=srcs_col, dsts_col=dsts_col,
                srcs_edge=src, dsts_edge=dst, z16=z16)


def _gcn_conv(x, W, b, n, aux):
    dinv = aux["dinv"]
    n_pad = aux["n_pad"]
    h = x @ W
    hp = dinv[:, None] * h
    w_out = W.shape[1]
    if w_out == 64:
        h2a = jnp.concatenate([hp[:, 0:16], hp[:, 16:32]], axis=0)
        h2b = jnp.concatenate([hp[:, 32:48], hp[:, 48:64]], axis=0)
        o2a = _sc_seg_sum(h2a, aux["srcs_col"], aux["dsts_col"], aux["z16"])
        o2b = _sc_seg_sum(h2b, aux["srcs_col"], aux["dsts_col"], aux["z16"])
        out0 = jnp.concatenate([o2a[:n], o2a[n_pad:n_pad + n],
                                o2b[:n], o2b[n_pad:n_pad + n]], axis=1)
    else:
        hp_pad = jnp.pad(hp, ((0, 0), (0, 16 - w_out)))
        o2 = _sc_seg_sum(hp_pad, aux["srcs_edge"], aux["dsts_edge"], aux["z16"])
        out0 = (o2[:n] + o2[n_pad:n_pad + n])[:, :w_out]
    return dinv[:, None] * out0 + (dinv * dinv)[:, None] * h + b


def _bn_relu_add_block(h_ref, stats_ref, idn_ref, out_ref):
    scale = stats_ref[0, :]
    shift = stats_ref[1, :]
    h = h_ref[...] * scale[None, :] + shift[None, :]
    out_ref[...] = jnp.maximum(h, 0.0) + idn_ref[...]


def _bn_relu_add(h, identity, gamma, beta, eps=1e-5):
    n, c = h.shape
    mu = jnp.mean(h, axis=0)
    var = jnp.mean((h - mu) ** 2, axis=0)
    rstd = 1.0 / jnp.sqrt(var + eps)
    stats = jnp.stack([gamma * rstd, beta - gamma * mu * rstd])
    blk = 1000
    grid = (n + blk - 1) // blk
    return pl.pallas_call(
        _bn_relu_add_block,
        out_shape=jax.ShapeDtypeStruct((n, c), h.dtype),
        grid=(grid,),
        in_specs=[
            pl.BlockSpec((blk, c), lambda i: (i, 0)),
            pl.BlockSpec((2, c), lambda i: (0, 0)),
            pl.BlockSpec((blk, c), lambda i: (i, 0)),
        ],
        out_specs=pl.BlockSpec((blk, c), lambda i: (i, 0)),
    )(h, stats, identity)


def _conv_res_block(x, p, n, aux):
    identity = x @ p["Wr"] + p["br"] if "Wr" in p else x
    h = _gcn_conv(x, p["Wg"], p["bg"], n, aux)
    return _bn_relu_add(h, identity, p["gamma"], p["beta"])


def _knn_block(posh_ref, plT_ref, feat_ref, out_ref, orig_ref, work_ref):
    qx = posh_ref[:, 0:1]
    qy = posh_ref[:, 1:2]
    px = plT_ref[0:1, :]
    py = plT_ref[1:2, :]
    dx = qx - px
    dy = qy - py
    d = dx * dx + dy * dy
    orig_ref[...] = d
    work_ref[...] = d

    def body(i, _):
        dw = work_ref[...]
        m = jnp.min(dw, axis=1, keepdims=True)
        work_ref[...] = jnp.where(dw <= m, jnp.float32(jnp.inf), dw)
        return m

    t = jax.lax.fori_loop(0, _K, body, jnp.zeros((posh_ref.shape[0], 1), jnp.float32))
    dorig = orig_ref[...]
    w = jnp.where(dorig <= t, 1.0 / jnp.maximum(dorig, 1e-16), 0.0)
    r = jnp.dot(w, feat_ref[...], preferred_element_type=jnp.float32,
                precision=jax.lax.Precision.HIGHEST)
    out_ref[...] = r / r[:, 6:7]


def _knn_interpolate(x, pos_l, pos_h, k=_K):
    n_h = pos_h.shape[0]
    n_l = pos_l.shape[0]
    npad = ((n_l + 127) // 128) * 128 + 128
    plT = jnp.full((8, npad), 1e9, dtype=jnp.float32)
    plT = plT.at[0, :n_l].set(pos_l[:, 0]).at[1, :n_l].set(pos_l[:, 1])
    feat = jnp.zeros((npad, 8), dtype=jnp.float32)
    feat = feat.at[:n_l, :x.shape[1]].set(x).at[:n_l, 6].set(1.0)
    qblk = 200
    grid = n_h // qblk
    out = pl.pallas_call(
        _knn_block,
        out_shape=jax.ShapeDtypeStruct((n_h, 8), jnp.float32),
        grid=(grid,),
        in_specs=[
            pl.BlockSpec((qblk, 2), lambda i: (i, 0)),
            pl.BlockSpec((8, npad), lambda i: (0, 0)),
            pl.BlockSpec((npad, 8), lambda i: (0, 0)),
        ],
        out_specs=pl.BlockSpec((qblk, 8), lambda i: (i, 0)),
        scratch_shapes=[
            pltpu.VMEM((qblk, npad), jnp.float32),
            pltpu.VMEM((qblk, npad), jnp.float32),
        ],
    )(pos_h, plT, feat)
    return out[:, :6]


def _cfd_error(pos_l, src, dst, p, n):
    x = _edge_conv(pos_l, src, dst, p["W0"], p["b0"], n)
    x1 = _edge_conv(x, src, dst, p["W1"], p["b1"], n)
    x2 = _edge_conv(x, src, dst, p["W2"], p["b2"], n)
    x2 = x2 * x2
    t1 = _edge_conv(x, src, dst, p["W3a"], p["b3a"], n)
    t2 = _edge_conv(x, src, dst, p["W3b"], p["b3b"], n)
    x4 = t1 * t1 + t2 * t2
    xc = jnp.concatenate([x1, x2, x4], axis=1)
    return _edge_conv(xc, src, dst, p["W4"], p["b4"], n)


def kernel(x_l, pos_l, pos_h, params, edge_index_l, edge_index_h):
    n_l = x_l.shape[0]
    n_h = pos_h.shape[0]
    sl, dl = edge_index_l[0], edge_index_l[1]
    sh, dh = edge_index_h[0], edge_index_h[1]
    e = _cfd_error(pos_l, sl, dl, params["err"], n_l)
    x = jnp.concatenate([x_l, e], axis=1)
    x = _knn_interpolate(x, pos_l, pos_h, _K)
    aux = _gcn_aux(sh, dh, n_h)
    x1 = _conv_res_block(x, params["enc1"], n_h, aux)
    x2 = _conv_res_block(x1, params["enc2"], n_h, aux)
    m = _gcn_conv(x2, params["mp"]["Wg"], params["mp"]["bg"], n_h, aux)
    dec_in = x1 + m
    d1 = _conv_res_block(dec_in, params["dec1"], n_h, aux)
    d2 = _conv_res_block(d1, params["dec2"], n_h, aux)
    return d2


# speedup vs baseline: 9.2671x; 1.1452x over previous
import functools

import jax
import jax.numpy as jnp
from jax import lax
from jax.experimental import pallas as pl
from jax.experimental.pallas import tpu as pltpu
from jax.experimental.pallas import tpu_sc as plsc

_K = 32


def _sc_seg_sum(table, srcs, dsts, zeros_half):
    w = table.shape[1]
    ec = srcs.shape[0] // 2
    ept = ec // 16
    ch = ept // 25
    n_pad = zeros_half.shape[0]
    stripe = n_pad // 16
    mesh = plsc.VectorSubcoreMesh(core_axis_name="c", subcore_axis_name="s")

    @functools.partial(
        pl.kernel, mesh=mesh,
        out_type=jax.ShapeDtypeStruct((2 * n_pad, w), jnp.float32),
        compiler_params=pltpu.CompilerParams(use_tc_tiling_on_sc=False),
        scratch_types=[
            pltpu.VMEM((ch,), jnp.int32),
            pltpu.VMEM((ch,), jnp.int32),
            pltpu.VMEM((ch, w), jnp.float32),
            pltpu.VMEM_SHARED((n_pad, w), jnp.float32),
            pltpu.SemaphoreType.DMA,
        ],
    )
    def body(t_hbm, s_hbm, d_hbm, z_hbm, out_hbm, sidx, didx, rows, acc, sem):
        cid = lax.axis_index("c")
        tid = lax.axis_index("s")
        pltpu.sync_copy(z_hbm.at[pl.ds(tid * stripe, stripe)],
                        acc.at[pl.ds(tid * stripe, stripe)])
        plsc.subcore_barrier()

        def step(i, carry):
            off = cid * ec + tid * ept + i * ch
            pltpu.sync_copy(s_hbm.at[pl.ds(off, ch)], sidx)
            pltpu.sync_copy(d_hbm.at[pl.ds(off, ch)], didx)
            pltpu.async_copy(t_hbm.at[sidx], rows, sem).wait()
            pltpu.sync_copy(rows, acc.at[didx], add=True)
            return carry

        lax.fori_loop(0, ept // ch, step, 0)
        plsc.subcore_barrier()
        pltpu.sync_copy(acc.at[pl.ds(tid * stripe, stripe)],
                        out_hbm.at[pl.ds(cid * n_pad + tid * stripe, stripe)])

    return body(table, srcs, dsts, zeros_half)


def _edge_conv_ab(A, B, src, dst, b, n):
    m = jax.ops.segment_max(B[src], dst, num_segments=n)
    return jnp.where(jnp.isneginf(m), 0.0, A + b + m)


def _gcn_aux(src, dst, n):
    deg = jnp.zeros((n,), jnp.float32).at[dst].add(1.0) + 1.0
    dinv = 1.0 / jnp.sqrt(deg)
    n_pad = ((n + 127) // 128) * 128
    srcs_col = jnp.concatenate([src, src + n])
    dsts_col = jnp.concatenate([dst, dst])
    z16 = jnp.zeros((n_pad, 16), jnp.float32)
    return dict(dinv=dinv, n_pad=n_pad, srcs_col=srcs_col, dsts_col=dsts_col,
                srcs_edge=src, dsts_edge=dst, z16=z16)


def _gcn_conv(x, W, b, n, aux):
    dinv = aux["dinv"]
    n_pad = aux["n_pad"]
    h = x @ W
    hp = dinv[:, None] * h
    w_out = W.shape[1]
    if w_out == 64:
        h2a = jnp.concatenate([hp[:, 0:16], hp[:, 16:32]], axis=0)
        h2b = jnp.concatenate([hp[:, 32:48], hp[:, 48:64]], axis=0)
        o2a = _sc_seg_sum(h2a, aux["srcs_col"], aux["dsts_col"], aux["z16"])
        o2b = _sc_seg_sum(h2b, aux["srcs_col"], aux["dsts_col"], aux["z16"])
        out0 = jnp.concatenate([o2a[:n], o2a[n_pad:n_pad + n],
                                o2b[:n], o2b[n_pad:n_pad + n]], axis=1)
    else:
        hp_pad = jnp.pad(hp, ((0, 0), (0, 16 - w_out)))
        o2 = _sc_seg_sum(hp_pad, aux["srcs_edge"], aux["dsts_edge"], aux["z16"])
        out0 = (o2[:n] + o2[n_pad:n_pad + n])[:, :w_out]
    return dinv[:, None] * out0 + (dinv * dinv)[:, None] * h + b


def _bn_relu_add_block(h_ref, stats_ref, idn_ref, out_ref):
    scale = stats_ref[0, :]
    shift = stats_ref[1, :]
    h = h_ref[...] * scale[None, :] + shift[None, :]
    out_ref[...] = jnp.maximum(h, 0.0) + idn_ref[...]


def _bn_relu_add(h, identity, gamma, beta, eps=1e-5):
    n, c = h.shape
    mu = jnp.mean(h, axis=0)
    var = jnp.mean((h - mu) ** 2, axis=0)
    rstd = 1.0 / jnp.sqrt(var + eps)
    stats = jnp.stack([gamma * rstd, beta - gamma * mu * rstd])
    blk = 1000
    grid = (n + blk - 1) // blk
    return pl.pallas_call(
        _bn_relu_add_block,
        out_shape=jax.ShapeDtypeStruct((n, c), h.dtype),
        grid=(grid,),
        in_specs=[
            pl.BlockSpec((blk, c), lambda i: (i, 0)),
            pl.BlockSpec((2, c), lambda i: (0, 0)),
            pl.BlockSpec((blk, c), lambda i: (i, 0)),
        ],
        out_specs=pl.BlockSpec((blk, c), lambda i: (i, 0)),
    )(h, stats, identity)


def _conv_res_block(x, p, n, aux):
    identity = x @ p["Wr"] + p["br"] if "Wr" in p else x
    h = _gcn_conv(x, p["Wg"], p["bg"], n, aux)
    return _bn_relu_add(h, identity, p["gamma"], p["beta"])


def _knn_block(posh_ref, plT_ref, feat_ref, out_ref, orig_ref, work_ref):
    qx = posh_ref[:, 0:1]
    qy = posh_ref[:, 1:2]
    px = plT_ref[0:1, :]
    py = plT_ref[1:2, :]
    dx = qx - px
    dy = qy - py
    d = dx * dx + dy * dy
    orig_ref[...] = d
    work_ref[...] = d

    def body(i, _):
        dw = work_ref[...]
        m = jnp.min(dw, axis=1, keepdims=True)
        work_ref[...] = jnp.where(dw <= m, jnp.float32(jnp.inf), dw)
        return m

    t = jax.lax.fori_loop(0, _K, body, jnp.zeros((posh_ref.shape[0], 1), jnp.float32))
    dorig = orig_ref[...]
    w = jnp.where(dorig <= t, 1.0 / jnp.maximum(dorig, 1e-16), 0.0)
    r = jnp.dot(w, feat_ref[...], preferred_element_type=jnp.float32,
                precision=jax.lax.Precision.HIGHEST)
    out_ref[...] = r / r[:, 6:7]


def _knn_interpolate(x, pos_l, pos_h, k=_K):
    n_h = pos_h.shape[0]
    n_l = pos_l.shape[0]
    npad = ((n_l + 127) // 128) * 128 + 128
    plT = jnp.full((8, npad), 1e9, dtype=jnp.float32)
    plT = plT.at[0, :n_l].set(pos_l[:, 0]).at[1, :n_l].set(pos_l[:, 1])
    feat = jnp.zeros((npad, 8), dtype=jnp.float32)
    feat = feat.at[:n_l, :x.shape[1]].set(x).at[:n_l, 6].set(1.0)
    qblk = 200
    grid = n_h // qblk
    out = pl.pallas_call(
        _knn_block,
        out_shape=jax.ShapeDtypeStruct((n_h, 8), jnp.float32),
        grid=(grid,),
        in_specs=[
            pl.BlockSpec((qblk, 2), lambda i: (i, 0)),
            pl.BlockSpec((8, npad), lambda i: (0, 0)),
            pl.BlockSpec((npad, 8), lambda i: (0, 0)),
        ],
        out_specs=pl.BlockSpec((qblk, 8), lambda i: (i, 0)),
        scratch_shapes=[
            pltpu.VMEM((qblk, npad), jnp.float32),
            pltpu.VMEM((qblk, npad), jnp.float32),
        ],
    )(pos_h, plT, feat)
    return out[:, :6]


def _cfd_error(pos_l, src, dst, p, n):
    def split(W):
        f = W.shape[0] // 2
        return W[:f] - W[f:], W[f:]
    W0t, W0b = split(p["W0"])
    x = _edge_conv_ab(pos_l @ W0t, pos_l @ W0b, src, dst, p["b0"], n)
    tops, bots = zip(*(split(p[k]) for k in ("W1", "W2", "W3a", "W3b")))
    A_cat = x @ jnp.concatenate(tops, axis=1)
    B_cat = x @ jnp.concatenate(bots, axis=1)
    b_cat = jnp.concatenate([p["b1"], p["b2"], p["b3a"], p["b3b"]])
    mid = _edge_conv_ab(A_cat, B_cat, src, dst, b_cat, n)
    x1, x2, t1, t2 = (mid[:, 0:64], mid[:, 64:128],
                      mid[:, 128:192], mid[:, 192:256])
    xc = jnp.concatenate([x1, x2 * x2, t1 * t1 + t2 * t2], axis=1)
    W4t, W4b = split(p["W4"])
    return _edge_conv_ab(xc @ W4t, xc @ W4b, src, dst, p["b4"], n)


def kernel(x_l, pos_l, pos_h, params, edge_index_l, edge_index_h):
    n_l = x_l.shape[0]
    n_h = pos_h.shape[0]
    sl, dl = edge_index_l[0], edge_index_l[1]
    sh, dh = edge_index_h[0], edge_index_h[1]
    e = _cfd_error(pos_l, sl, dl, params["err"], n_l)
    x = jnp.concatenate([x_l, e], axis=1)
    x = _knn_interpolate(x, pos_l, pos_h, _K)
    aux = _gcn_aux(sh, dh, n_h)
    x1 = _conv_res_block(x, params["enc1"], n_h, aux)
    x2 = _conv_res_block(x1, params["enc2"], n_h, aux)
    m = _gcn_conv(x2, params["mp"]["Wg"], params["mp"]["bg"], n_h, aux)
    dec_in = x1 + m
    d1 = _conv_res_block(dec_in, params["dec1"], n_h, aux)
    d2 = _conv_res_block(d1, params["dec2"], n_h, aux)
    return d2
